# Initial kernel scaffold; baseline (speedup 1.0000x reference)
import jax
import jax.numpy as jnp
from jax.experimental import pallas as pl


def kernel(position, table0, table1, table2):
    raise NotImplementedError("write your pallas kernel here")


# speedup vs baseline: 6.9328x; 6.9328x over previous
import functools

import jax
import jax.numpy as jnp
from jax import lax
from jax.experimental import pallas as pl
from jax.experimental.pallas import tpu as pltpu
from jax.experimental.pallas import tpu_sc as plsc

N, S, A = 1024, 200, 3
E = 128
NROWS = N * S
NC, NSUB = 2, 16
NW = NC * NSUB
ROWS_PER_W = NROWS // NW
G = 128
GPW = ROWS_PER_W // G


def _sc_body(t0, t1, t2, idx_hbm, out_hbm, idxv, buf, sem):
    c = lax.axis_index("c")
    s = lax.axis_index("s")
    wid = s * NC + c
    pltpu.sync_copy(idx_hbm.at[wid], idxv)
    tabs = (t0, t1, t2)

    def group(g, carry):
        cps = [
            pltpu.async_copy(tabs[a].at[idxv.at[a, g]], buf.at[a], sem)
            for a in range(3)
        ]
        for cp in cps:
            cp.wait()

        def row(r, carry2):
            for cc in range(E // 16):
                sl = pl.ds(cc * 16, 16)
                buf[0, r, sl] = buf[0, r, sl] + buf[1, r, sl] + buf[2, r, sl]
            return carry2

        lax.fori_loop(0, G, row, 0)
        base = (wid * GPW + g) * G
        pltpu.sync_copy(buf.at[0], out_hbm.at[pl.ds(base, G)])
        return carry

    lax.fori_loop(0, GPW, group, 0)


_mesh = plsc.VectorSubcoreMesh(
    core_axis_name="c", subcore_axis_name="s", num_cores=NC, num_subcores=NSUB
)

_call = functools.partial(
    pl.kernel,
    out_type=jax.ShapeDtypeStruct((NROWS, E), jnp.float32),
    mesh=_mesh,
    scratch_types=[
        pltpu.VMEM((A, GPW, G), jnp.int32),
        pltpu.VMEM((A, G, E), jnp.float32),
        pltpu.SemaphoreType.DMA,
    ],
)(_sc_body)


def kernel(position, table0, table1, table2):
    idx = position.reshape(NROWS, A).T.reshape(A, NW, GPW, G)
    idx = idx.transpose(1, 0, 2, 3)
    out = _call(table0, table1, table2, idx)
    return out.reshape(N, S, E)


# speedup vs baseline: 9.9980x; 1.4421x over previous
import functools

import jax
import jax.numpy as jnp
from jax import lax
from jax.experimental import pallas as pl
from jax.experimental.pallas import tpu as pltpu
from jax.experimental.pallas import tpu_sc as plsc

N, S, A = 1024, 200, 3
E = 128
NROWS = N * S
NC, NSUB = 2, 16
NW = NC * NSUB
ROWS_PER_W = NROWS // NW
G = 128
GPW = ROWS_PER_W // G


def _sc_body(t0, t1, t2, idx_hbm, out_hbm, idxv, buf, sem0, sem1):
    c = lax.axis_index("c")
    s = lax.axis_index("s")
    wid = s * NC + c
    pltpu.sync_copy(idx_hbm.at[wid], idxv)
    tabs = (t0, t1, t2)
    sems = (sem0, sem1)

    def issue(g, p):
        for a in range(3):
            pltpu.async_copy(tabs[a].at[idxv.at[a, g]], buf.at[p, a], sems[p])

    def wait(g, p):
        for a in range(3):
            pltpu.make_async_copy(
                tabs[a].at[idxv.at[a, g]], buf.at[p, a], sems[p]
            ).wait()

    def compute(p):
        def row(r, carry):
            for cc in range(E // 16):
                sl = pl.ds(cc * 16, 16)
                buf[p, 0, r, sl] = (
                    buf[p, 0, r, sl] + buf[p, 1, r, sl] + buf[p, 2, r, sl]
                )
            return carry

        lax.fori_loop(0, G, row, 0)

    def out(g, p):
        base = (wid * GPW + g) * G
        pltpu.sync_copy(buf.at[p, 0], out_hbm.at[pl.ds(base, G)])

    issue(0, 0)

    def pair(i, carry):
        g = 2 * i
        issue(g + 1, 1)
        wait(g, 0)
        compute(0)
        out(g, 0)
        issue(g + 2, 0)
        wait(g + 1, 1)
        compute(1)
        out(g + 1, 1)
        return carry

    lax.fori_loop(0, GPW // 2 - 1, pair, 0)
    ge = GPW - 2
    issue(ge + 1, 1)
    wait(ge, 0)
    compute(0)
    out(ge, 0)
    wait(ge + 1, 1)
    compute(1)
    out(ge + 1, 1)


_mesh = plsc.VectorSubcoreMesh(
    core_axis_name="c", subcore_axis_name="s", num_cores=NC, num_subcores=NSUB
)

_call = functools.partial(
    pl.kernel,
    out_type=jax.ShapeDtypeStruct((NROWS, E), jnp.float32),
    mesh=_mesh,
    scratch_types=[
        pltpu.VMEM((A, GPW, G), jnp.int32),
        pltpu.VMEM((2, A, G, E), jnp.float32),
        pltpu.SemaphoreType.DMA,
        pltpu.SemaphoreType.DMA,
    ],
)(_sc_body)


def kernel(position, table0, table1, table2):
    idx = position.reshape(NROWS, A).T.reshape(A, NW, GPW, G)
    idx = idx.transpose(1, 0, 2, 3)
    out = _call(table0, table1, table2, idx)
    return out.reshape(N, S, E)


# speedup vs baseline: 10.1963x; 1.0198x over previous
import functools

import jax
import jax.numpy as jnp
from jax import lax
from jax.experimental import pallas as pl
from jax.experimental.pallas import tpu as pltpu
from jax.experimental.pallas import tpu_sc as plsc

N, S, A = 1024, 200, 3
E = 128
NROWS = N * S
NC, NSUB = 2, 16
NW = NC * NSUB
ROWS_PER_W = NROWS // NW
G = 128
GPW = ROWS_PER_W // G


def _sc_body(t0, t1, t2, idx_hbm, out_hbm, idxv, buf, sem0, sem1):
    c = lax.axis_index("c")
    s = lax.axis_index("s")
    wid = s * NC + c
    pltpu.sync_copy(idx_hbm.at[wid], idxv)
    tabs = (t0, t1, t2)
    sems = (sem0, sem1)

    def zero(p):
        z = jnp.zeros((16,), jnp.float32)

        def row(r, carry):
            for cc in range(E // 16):
                buf[p, r, pl.ds(cc * 16, 16)] = z
            return carry

        lax.fori_loop(0, G, row, 0)

    def issue(g, p):
        for a in range(3):
            pltpu.async_copy(
                tabs[a].at[idxv.at[a, g]], buf.at[p], sems[p], add=True
            )

    def wait(g, p):
        for a in range(3):
            pltpu.make_async_copy(
                tabs[a].at[idxv.at[a, g]], buf.at[p], sems[p]
            ).wait()

    def out(g, p):
        base = (wid * GPW + g) * G
        pltpu.sync_copy(buf.at[p], out_hbm.at[pl.ds(base, G)])

    zero(0)
    issue(0, 0)

    def pair(i, carry):
        g = 2 * i
        zero(1)
        issue(g + 1, 1)
        wait(g, 0)
        out(g, 0)
        zero(0)
        issue(g + 2, 0)
        wait(g + 1, 1)
        out(g + 1, 1)
        return carry

    lax.fori_loop(0, GPW // 2 - 1, pair, 0)
    ge = GPW - 2
    zero(1)
    issue(ge + 1, 1)
    wait(ge, 0)
    out(ge, 0)
    wait(ge + 1, 1)
    out(ge + 1, 1)


_mesh = plsc.VectorSubcoreMesh(
    core_axis_name="c", subcore_axis_name="s", num_cores=NC, num_subcores=NSUB
)

_call = functools.partial(
    pl.kernel,
    out_type=jax.ShapeDtypeStruct((NROWS, E), jnp.float32),
    mesh=_mesh,
    scratch_types=[
        pltpu.VMEM((A, GPW, G), jnp.int32),
        pltpu.VMEM((2, G, E), jnp.float32),
        pltpu.SemaphoreType.DMA,
        pltpu.SemaphoreType.DMA,
    ],
)(_sc_body)


def kernel(position, table0, table1, table2):
    idx = position.reshape(NROWS, A).T.reshape(A, NW, GPW, G)
    idx = idx.transpose(1, 0, 2, 3)
    out = _call(table0, table1, table2, idx)
    return out.reshape(N, S, E)
